# baseline (speedup 1.0000x reference)
import functools

import jax
import jax.numpy as jnp
from jax import lax
from jax.experimental import pallas as pl
from jax.experimental.pallas import tpu as pltpu
from jax.experimental.pallas import tpu_sc as plsc

MAXLEN = 200
N_ROWS = 1024
N_COLS = 128
NWORDS = MAXLEN * N_ROWS * N_COLS
XWORDS = N_ROWS * N_COLS
ZWORDS = NWORDS - XWORDS
NW = 32
PER_W = ZWORDS // NW
ZBUF = 32768
NFULL = PER_W // ZBUF
TAIL = PER_W - NFULL * ZBUF


def _sc_body(x_hbm, out_hbm, zbuf, sem):
    wid = lax.axis_index("s") * 2 + lax.axis_index("c")

    def _zero(i, carry):
        zbuf[pl.ds(i * 16, 16)] = jnp.zeros((16,), jnp.float32)
        return carry

    lax.fori_loop(0, ZBUF // 16, _zero, 0)

    base = wid * PER_W
    copies = []
    for k in range(NFULL):
        copies.append(
            pltpu.make_async_copy(
                zbuf, out_hbm.at[pl.ds(base + k * ZBUF, ZBUF)], sem
            )
        )
    if TAIL:
        copies.append(
            pltpu.make_async_copy(
                zbuf.at[pl.ds(0, TAIL)],
                out_hbm.at[pl.ds(base + NFULL * ZBUF, TAIL)],
                sem,
            )
        )
    for c in copies:
        c.start()

    @pl.when(wid == 0)
    def _():
        pltpu.sync_copy(x_hbm, out_hbm.at[pl.ds(ZWORDS, XWORDS)])

    for c in copies:
        c.wait()


def kernel(input):
    mesh = plsc.VectorSubcoreMesh(core_axis_name="c", subcore_axis_name="s")
    k = functools.partial(
        pl.kernel,
        mesh=mesh,
        out_type=jax.ShapeDtypeStruct((NWORDS,), jnp.float32),
        scratch_types=[
            pltpu.VMEM((ZBUF,), jnp.float32),
            pltpu.SemaphoreType.DMA,
        ],
    )(_sc_body)
    out = k(input.reshape(XWORDS))
    return out.reshape(MAXLEN, N_ROWS, N_COLS)


# speedup vs baseline: 1.9051x; 1.9051x over previous
import jax
import jax.numpy as jnp
from jax.experimental import pallas as pl
from jax.experimental.pallas import tpu as pltpu

MAXLEN = 200
ZBLOCK = 1


def _fill_body(x_ref, o_ref, zbuf, sem):
    zbuf[...] = jnp.zeros_like(zbuf)
    nfull = (MAXLEN - 1) // ZBLOCK
    tail = MAXLEN - 1 - nfull * ZBLOCK
    copies = []
    for i in range(nfull):
        copies.append(
            pltpu.make_async_copy(zbuf, o_ref.at[pl.ds(i * ZBLOCK, ZBLOCK)], sem)
        )
    if tail:
        copies.append(
            pltpu.make_async_copy(
                zbuf.at[pl.ds(0, tail)], o_ref.at[pl.ds(nfull * ZBLOCK, tail)], sem
            )
        )
    copies.append(pltpu.make_async_copy(x_ref, o_ref.at[pl.ds(MAXLEN - 1, 1)], sem))
    for c in copies:
        c.start()
    for c in copies:
        c.wait()


def kernel(input):
    n, d = input.shape
    return pl.pallas_call(
        _fill_body,
        in_specs=[pl.BlockSpec(memory_space=pl.ANY)],
        out_specs=pl.BlockSpec(memory_space=pl.ANY),
        out_shape=jax.ShapeDtypeStruct((MAXLEN, n, d), input.dtype),
        scratch_shapes=[
            pltpu.VMEM((ZBLOCK, n, d), input.dtype),
            pltpu.SemaphoreType.DMA,
        ],
    )(input.reshape(1, n, d))


# speedup vs baseline: 1.9424x; 1.0196x over previous
import jax
import jax.numpy as jnp
from jax.experimental import pallas as pl
from jax.experimental.pallas import tpu as pltpu

MAXLEN = 200
ZBLOCK = 2


def _fill_body(x_ref, o_ref, zbuf, sem):
    zbuf[...] = jnp.zeros_like(zbuf)
    nfull = (MAXLEN - 1) // ZBLOCK
    tail = MAXLEN - 1 - nfull * ZBLOCK
    copies = []
    for i in range(nfull):
        copies.append(
            pltpu.make_async_copy(zbuf, o_ref.at[pl.ds(i * ZBLOCK, ZBLOCK)], sem)
        )
    if tail:
        copies.append(
            pltpu.make_async_copy(
                zbuf.at[pl.ds(0, tail)], o_ref.at[pl.ds(nfull * ZBLOCK, tail)], sem
            )
        )
    copies.insert(0, pltpu.make_async_copy(x_ref, o_ref.at[pl.ds(MAXLEN - 1, 1)], sem))
    for c in copies:
        c.start()
    for c in copies:
        c.wait()


def kernel(input):
    n, d = input.shape
    return pl.pallas_call(
        _fill_body,
        in_specs=[pl.BlockSpec(memory_space=pl.ANY)],
        out_specs=pl.BlockSpec(memory_space=pl.ANY),
        out_shape=jax.ShapeDtypeStruct((MAXLEN, n, d), input.dtype),
        scratch_shapes=[
            pltpu.VMEM((ZBLOCK, n, d), input.dtype),
            pltpu.SemaphoreType.DMA,
        ],
    )(input.reshape(1, n, d))
